# baseline (speedup 1.0000x reference)
import functools

import jax
import jax.numpy as jnp
from jax import lax
from jax.experimental import pallas as pl
from jax.experimental.pallas import tpu as pltpu
from jax.experimental.pallas import tpu_sc as plsc

_BATCH = 16384
_FIELDS = 100
_LANES = 16
_NUM_WORKERS = 32
_ROWS_W = _BATCH // _NUM_WORKERS
_FULL = _FIELDS // _LANES
_TAIL_OFF = _FIELDS - _LANES
_TAIL_DUP = _LANES - (_FIELDS - _FULL * _LANES)
_G = 8
_NGROUPS = _ROWS_W // _G
_SLOTS = 2 * _G

_mesh = plsc.VectorSubcoreMesh(core_axis_name="c", subcore_axis_name="s")


@functools.partial(
    pl.kernel,
    mesh=_mesh,
    out_type=jax.ShapeDtypeStruct((_NUM_WORKERS, _LANES), jnp.float32),
    scratch_types=[
        pltpu.VMEM((_ROWS_W, _FIELDS), jnp.int32),
        pltpu.VMEM((_SLOTS, _FIELDS), jnp.float32),
        pltpu.VMEM((_LANES,), jnp.float32),
        pltpu.SemaphoreType.DMA,
        pltpu.SemaphoreType.DMA,
    ] + [pltpu.SemaphoreType.DMA] * _SLOTS,
)
def _gather_sum(idx_hbm, w_hbm, out_hbm, idx_v, vals_v, acc_v,
                isem0, isem1, *gsems):
    nc = plsc.get_sparse_core_info().num_cores
    wid = lax.axis_index("s") * nc + lax.axis_index("c")
    row0 = wid * _ROWS_W
    half = _ROWS_W // 2

    h0 = pltpu.async_copy(
        idx_hbm.at[pl.ds(row0, half), :], idx_v.at[pl.ds(0, half), :], isem0)
    h1 = pltpu.async_copy(
        idx_hbm.at[pl.ds(row0 + half, half), :],
        idx_v.at[pl.ds(half, half), :], isem1)
    h0.wait()

    def issue(g, p):
        for j in range(_G):
            pltpu.async_copy(
                w_hbm.at[idx_v.at[g * _G + j]],
                vals_v.at[p * _G + j], gsems[p * _G + j])

    tail_mask = lax.iota(jnp.int32, _LANES) < _TAIL_DUP
    fzero = jnp.zeros((_LANES,), jnp.float32)

    def drain_reduce(g, p, accs):
        for j in range(_G):
            s = p * _G + j
            pltpu.make_async_copy(
                w_hbm.at[idx_v.at[g * _G + j]], vals_v.at[s], gsems[s]).wait()
        new = list(accs)
        for j in range(_G):
            s = p * _G + j
            for k in range(_FULL):
                new[k] = new[k] + vals_v[s, pl.ds(k * _LANES, _LANES)]
            tail = vals_v[s, pl.ds(_TAIL_OFF, _LANES)]
            new[_FULL] = new[_FULL] + jnp.where(tail_mask, fzero, tail)
        return tuple(new)

    issue(0, 0)
    issue(1, 1)

    def steady(g2, accs):
        g = 2 * g2
        accs = drain_reduce(g, 0, accs)
        issue(g + 2, 0)
        accs = drain_reduce(g + 1, 1, accs)
        issue(g + 3, 1)
        return accs

    accs = (fzero,) * (_FULL + 1)
    accs = lax.fori_loop(0, _NGROUPS // 4 - 1, steady, accs)
    h1.wait()
    accs = lax.fori_loop(_NGROUPS // 4 - 1, _NGROUPS // 2 - 1, steady, accs)
    accs = drain_reduce(_NGROUPS - 2, 0, accs)
    accs = drain_reduce(_NGROUPS - 1, 1, accs)

    total = accs[0]
    for j in range(1, _FULL + 1):
        total = total + accs[j]
    acc_v[...] = total
    pltpu.sync_copy(acc_v, out_hbm.at[wid])


def kernel(non_zero_indices, weights):
    partials = _gather_sum(non_zero_indices, weights)
    return jnp.sum(partials)
